# baseline (speedup 1.0000x reference)
import functools

import jax
import jax.numpy as jnp
from jax import lax
from jax.experimental import pallas as pl
from jax.experimental.pallas import tpu as pltpu
from jax.experimental.pallas import tpu_sc as plsc

_B, _T, _D = 4, 16384, 128
_NC, _NS = 2, 16
_NW = _NC * _NS
_TPW = _T // _NW
_C = 128
_NCHUNK = _TPW // _C


def _sc_body(enc_hbm, tab_hbm, pos_hbm, out_hbm, idx_v, rows_v, enc_v, gsem):
    wid = lax.axis_index("s") * _NC + lax.axis_index("c")
    t0 = wid * _TPW

    def chunk(ci, carry):
        tc0 = t0 + ci * _C
        pltpu.sync_copy(pos_hbm.at[pl.ds(tc0, _C)], idx_v)
        pltpu.async_copy(tab_hbm.at[idx_v], rows_v, gsem).wait()
        for b in range(_B):
            pltpu.sync_copy(enc_hbm.at[b, pl.ds(tc0, _C)], enc_v)

            @plsc.parallel_loop(0, _C, unroll=4)
            def _row(i):
                for j in range(_D // 16):
                    s = pl.ds(j * 16, 16)
                    enc_v[i, s] = enc_v[i, s] + rows_v[i, s]

            pltpu.sync_copy(enc_v, out_hbm.at[b, pl.ds(tc0, _C)])
        return carry

    lax.fori_loop(0, _NCHUNK, chunk, 0)


@functools.partial(jax.jit, static_argnames=())
def kernel(encoded_tokens, position_table, positions):
    mesh = plsc.VectorSubcoreMesh(
        core_axis_name="c", subcore_axis_name="s",
        num_cores=_NC, num_subcores=_NS,
    )
    run = pl.kernel(
        _sc_body,
        out_type=jax.ShapeDtypeStruct((_B, _T, _D), jnp.float32),
        mesh=mesh,
        scratch_types=[
            pltpu.VMEM((_C,), jnp.int32),
            pltpu.VMEM((_C, _D), jnp.float32),
            pltpu.VMEM((_C, _D), jnp.float32),
            pltpu.SemaphoreType.DMA,
        ],
    )
    return run(encoded_tokens, position_table, positions)


# speedup vs baseline: 1.5757x; 1.5757x over previous
import jax
import jax.numpy as jnp
from jax import lax
from jax.experimental import pallas as pl
from jax.experimental.pallas import tpu as pltpu
from jax.experimental.pallas import tpu_sc as plsc

_B, _T, _D = 4, 16384, 128
_NC, _NS = 2, 16
_NW = _NC * _NS
_TPW = _T // _NW
_C = 64
_NCHUNK = _TPW // _C
_J = _D // 16


def _sc_body(enc_hbm, tab_hbm, pos_hbm, out_hbm, idx_v, rows_v, enc_v, gsem, esem, osem):
    wid = lax.axis_index("s") * _NC + lax.axis_index("c")
    t0 = wid * _TPW
    pltpu.sync_copy(pos_hbm.at[pl.ds(t0, _TPW)], idx_v)

    def start_gather(ci):
        return pltpu.async_copy(
            tab_hbm.at[idx_v.at[pl.ds(ci * _C, _C)]], rows_v.at[ci % 2], gsem)

    def start_enc_in(ci):
        tc0 = t0 + ci * _C
        return [pltpu.async_copy(enc_hbm.at[b, pl.ds(tc0, _C)],
                                 enc_v.at[ci % 2, b], esem)
                for b in range(_B)]

    def start_out(ci):
        tc0 = t0 + ci * _C
        return [pltpu.async_copy(enc_v.at[ci % 2, b],
                                 out_hbm.at[b, pl.ds(tc0, _C)], osem)
                for b in range(_B)]

    g_d = start_gather(0)
    e_d = start_enc_in(0)
    o_d = None
    for ci in range(_NCHUNK):
        cur = ci % 2
        if o_d is not None:
            for d in o_d:
                d.wait()
            o_d = None
        g_next, e_next = None, None
        if ci + 1 < _NCHUNK:
            g_next = start_gather(ci + 1)
            e_next = start_enc_in(ci + 1)
        g_d.wait()
        for d in e_d:
            d.wait()

        @plsc.parallel_loop(0, _C)
        def _row(i):
            r = [rows_v[cur, i, pl.ds(j * 16, 16)] for j in range(_J)]
            for b in range(_B):
                for j in range(_J):
                    s = pl.ds(j * 16, 16)
                    enc_v[cur, b, i, s] = enc_v[cur, b, i, s] + r[j]

        out_d = start_out(ci)
        if ci + 1 < _NCHUNK:
            g_d, e_d = g_next, e_next
            o_d = out_d
        else:
            for d in out_d:
                d.wait()


def kernel(encoded_tokens, position_table, positions):
    mesh = plsc.VectorSubcoreMesh(
        core_axis_name="c", subcore_axis_name="s",
        num_cores=_NC, num_subcores=_NS,
    )
    run = pl.kernel(
        _sc_body,
        out_type=jax.ShapeDtypeStruct((_B, _T, _D), jnp.float32),
        mesh=mesh,
        scratch_types=[
            pltpu.VMEM((_TPW,), jnp.int32),
            pltpu.VMEM((2, _C, _D), jnp.float32),
            pltpu.VMEM((2, _B, _C, _D), jnp.float32),
            pltpu.SemaphoreType.DMA,
            pltpu.SemaphoreType.DMA,
            pltpu.SemaphoreType.DMA,
        ],
    )
    return run(encoded_tokens, position_table, positions)


# speedup vs baseline: 1.6028x; 1.0172x over previous
import jax
import jax.numpy as jnp
from jax import lax
from jax.experimental import pallas as pl
from jax.experimental.pallas import tpu as pltpu
from jax.experimental.pallas import tpu_sc as plsc

_B, _T, _D = 4, 16384, 128
_NC, _NS = 2, 16
_NW = _NC * _NS
_TPW = _T // _NW
_C = 64
_NCHUNK = _TPW // _C
_J = _D // 16


def _sc_body(enc_hbm, tab_hbm, pos_hbm, out_hbm, idx_v, rows_v, enc_v, gsem, esem, osem):
    wid = lax.axis_index("s") * _NC + lax.axis_index("c")
    t0 = wid * _TPW
    pltpu.sync_copy(pos_hbm.at[pl.ds(t0, _TPW)], idx_v)

    def start_gather(ci):
        return pltpu.async_copy(
            tab_hbm.at[idx_v.at[pl.ds(ci * _C, _C)]], rows_v.at[ci % 2], gsem)

    def start_enc_in(ci):
        tc0 = t0 + ci * _C
        return [pltpu.async_copy(enc_hbm.at[b, pl.ds(tc0, _C)],
                                 enc_v.at[ci % 2, b], esem)
                for b in range(_B)]

    def start_out(ci):
        tc0 = t0 + ci * _C
        return [pltpu.async_copy(enc_v.at[ci % 2, b],
                                 out_hbm.at[b, pl.ds(tc0, _C)], osem)
                for b in range(_B)]

    g_d = start_gather(0)
    e_d = start_enc_in(0)
    o_d = None
    for ci in range(_NCHUNK):
        cur = ci % 2
        if o_d is not None:
            for d in o_d:
                d.wait()
            o_d = None
        g_next, e_next = None, None
        if ci + 1 < _NCHUNK:
            g_next = start_gather(ci + 1)
            e_next = start_enc_in(ci + 1)
        g_d.wait()
        for d in e_d:
            d.wait()

        @plsc.parallel_loop(0, _C)
        def _row(i):
            r = [rows_v[cur, i, pl.ds(j * 16, 16)] for j in range(_J)]
            for b in range(_B):
                for j in range(_J):
                    s = pl.ds(j * 16, 16)
                    plsc.addupdate(enc_v.at[cur, b, i, s], r[j])

        out_d = start_out(ci)
        if ci + 1 < _NCHUNK:
            g_d, e_d = g_next, e_next
            o_d = out_d
        else:
            for d in out_d:
                d.wait()


def kernel(encoded_tokens, position_table, positions):
    mesh = plsc.VectorSubcoreMesh(
        core_axis_name="c", subcore_axis_name="s",
        num_cores=_NC, num_subcores=_NS,
    )
    run = pl.kernel(
        _sc_body,
        out_type=jax.ShapeDtypeStruct((_B, _T, _D), jnp.float32),
        mesh=mesh,
        scratch_types=[
            pltpu.VMEM((_TPW,), jnp.int32),
            pltpu.VMEM((2, _C, _D), jnp.float32),
            pltpu.VMEM((2, _B, _C, _D), jnp.float32),
            pltpu.SemaphoreType.DMA,
            pltpu.SemaphoreType.DMA,
            pltpu.SemaphoreType.DMA,
        ],
    )
    return run(encoded_tokens, position_table, positions)


# speedup vs baseline: 1.6282x; 1.0159x over previous
import jax
import jax.numpy as jnp
from jax import lax
from jax.experimental import pallas as pl
from jax.experimental.pallas import tpu as pltpu
from jax.experimental.pallas import tpu_sc as plsc

_B, _T, _D = 4, 16384, 128
_NC, _NS = 2, 16
_NW = _NC * _NS
_TPW = _T // _NW
_C = 64
_NCHUNK = _TPW // _C
_J = _D // 16
_DEPTH = 3


def _sc_body(enc_hbm, tab_hbm, pos_hbm, out_hbm, idx_v, rows_v, enc_v, gsem, esem, osem):
    wid = lax.axis_index("s") * _NC + lax.axis_index("c")
    t0 = wid * _TPW
    pltpu.sync_copy(pos_hbm.at[pl.ds(t0, _TPW)], idx_v)

    def start_gather(ci):
        return pltpu.async_copy(
            tab_hbm.at[idx_v.at[pl.ds(ci * _C, _C)]], rows_v.at[ci % _DEPTH], gsem)

    def start_enc_in(ci):
        tc0 = t0 + ci * _C
        return [pltpu.async_copy(enc_hbm.at[b, pl.ds(tc0, _C)],
                                 enc_v.at[ci % _DEPTH, b], esem)
                for b in range(_B)]

    def start_out(ci):
        tc0 = t0 + ci * _C
        return [pltpu.async_copy(enc_v.at[ci % _DEPTH, b],
                                 out_hbm.at[b, pl.ds(tc0, _C)], osem)
                for b in range(_B)]

    g_d = [start_gather(ci) for ci in range(2)]
    e_d = [start_enc_in(ci) for ci in range(2)]
    o_d = [None] * _NCHUNK
    for ci in range(_NCHUNK):
        cur = ci % _DEPTH
        if ci >= 2:
            for d in o_d[ci - 2]:
                d.wait()
        if ci + 1 < _NCHUNK and ci >= 1:
            g_d.append(start_gather(ci + 1))
            e_d.append(start_enc_in(ci + 1))
        g_d[ci].wait()
        for d in e_d[ci]:
            d.wait()

        @plsc.parallel_loop(0, _C)
        def _row(i):
            r = [rows_v[cur, i, pl.ds(j * 16, 16)] for j in range(_J)]
            for b in range(_B):
                for j in range(_J):
                    s = pl.ds(j * 16, 16)
                    plsc.addupdate(enc_v.at[cur, b, i, s], r[j])

        o_d[ci] = start_out(ci)
    for ci in (_NCHUNK - 2, _NCHUNK - 1):
        for d in o_d[ci]:
            d.wait()


def kernel(encoded_tokens, position_table, positions):
    mesh = plsc.VectorSubcoreMesh(
        core_axis_name="c", subcore_axis_name="s",
        num_cores=_NC, num_subcores=_NS,
    )
    run = pl.kernel(
        _sc_body,
        out_type=jax.ShapeDtypeStruct((_B, _T, _D), jnp.float32),
        mesh=mesh,
        scratch_types=[
            pltpu.VMEM((_TPW,), jnp.int32),
            pltpu.VMEM((_DEPTH, _C, _D), jnp.float32),
            pltpu.VMEM((_DEPTH, _B, _C, _D), jnp.float32),
            pltpu.SemaphoreType.DMA,
            pltpu.SemaphoreType.DMA,
            pltpu.SemaphoreType.DMA,
        ],
    )
    return run(encoded_tokens, position_table, positions)


# speedup vs baseline: 1.6566x; 1.0174x over previous
import jax
import jax.numpy as jnp
from jax import lax
from jax.experimental import pallas as pl
from jax.experimental.pallas import tpu as pltpu
from jax.experimental.pallas import tpu_sc as plsc

_B, _T, _D = 4, 16384, 128
_NC, _NS = 2, 16
_NW = _NC * _NS
_TPW = _T // _NW
_C = 64
_NCHUNK = _TPW // _C
_J = _D // 16
_DEPTH = 3


def _sc_body(enc_hbm, tab_hbm, pos_hbm, out_hbm, idx_v, rows_v, enc_v, gsem, esem, osem):
    wid = lax.axis_index("s") * _NC + lax.axis_index("c")
    t0 = wid * _TPW
    pltpu.sync_copy(pos_hbm.at[pl.ds(t0, _TPW)], idx_v)

    def start_gather(ci):
        return pltpu.async_copy(
            tab_hbm.at[idx_v.at[pl.ds(ci * _C, _C)]], rows_v.at[ci % _DEPTH], gsem)

    def start_enc_in(ci):
        tc0 = t0 + ci * _C
        return [pltpu.async_copy(enc_hbm.at[pl.ds(0, _B), pl.ds(tc0, _C)],
                                 enc_v.at[ci % _DEPTH], esem)]

    def start_out(ci):
        tc0 = t0 + ci * _C
        return [pltpu.async_copy(enc_v.at[ci % _DEPTH],
                                 out_hbm.at[pl.ds(0, _B), pl.ds(tc0, _C)], osem)]

    g_d = [start_gather(ci) for ci in range(2)]
    e_d = [start_enc_in(ci) for ci in range(2)]
    o_d = [None] * _NCHUNK
    for ci in range(_NCHUNK):
        cur = ci % _DEPTH
        if ci >= 2:
            for d in o_d[ci - 2]:
                d.wait()
        if ci + 1 < _NCHUNK and ci >= 1:
            g_d.append(start_gather(ci + 1))
            e_d.append(start_enc_in(ci + 1))
        g_d[ci].wait()
        for d in e_d[ci]:
            d.wait()

        @plsc.parallel_loop(0, _C)
        def _row(i):
            r = [rows_v[cur, i, pl.ds(j * 16, 16)] for j in range(_J)]
            for b in range(_B):
                for j in range(_J):
                    s = pl.ds(j * 16, 16)
                    plsc.addupdate(enc_v.at[cur, b, i, s], r[j])

        o_d[ci] = start_out(ci)
    for ci in (_NCHUNK - 2, _NCHUNK - 1):
        for d in o_d[ci]:
            d.wait()


def kernel(encoded_tokens, position_table, positions):
    mesh = plsc.VectorSubcoreMesh(
        core_axis_name="c", subcore_axis_name="s",
        num_cores=_NC, num_subcores=_NS,
    )
    run = pl.kernel(
        _sc_body,
        out_type=jax.ShapeDtypeStruct((_B, _T, _D), jnp.float32),
        mesh=mesh,
        scratch_types=[
            pltpu.VMEM((_TPW,), jnp.int32),
            pltpu.VMEM((_DEPTH, _C, _D), jnp.float32),
            pltpu.VMEM((_DEPTH, _B, _C, _D), jnp.float32),
            pltpu.SemaphoreType.DMA,
            pltpu.SemaphoreType.DMA,
            pltpu.SemaphoreType.DMA,
        ],
    )
    return run(encoded_tokens, position_table, positions)
